# baseline (speedup 1.0000x reference)
import functools

import jax
import jax.numpy as jnp
import numpy as np
from jax.experimental import pallas as pl
from jax.experimental.pallas import tpu as pltpu

B = 8
N5 = 512
N10 = 1024
TOP = 256
L1 = 256
R4 = 4

_k5 = np.arange(N5)
_a5 = 2.0 * np.pi * ((_k5[:, None] * _k5[None, :]) % N5) / N5
_C5 = np.cos(_a5).astype(np.float32)
_S5 = (-np.sin(_a5)).astype(np.float32)

_l = np.arange(L1)
_aB = 2.0 * np.pi * ((_l[:, None] * _l[None, :]) % L1) / L1
_CB = (np.cos(_aB) / N10).astype(np.float32)
_SB = (np.sin(_aB) / N10).astype(np.float32)

_r = np.arange(R4)
_aTB = 2.0 * np.pi * ((_r[:, None] * _l[None, :]) % N10) / N10
_TWBR = np.cos(_aTB).astype(np.float32)
_TWBI = np.sin(_aTB).astype(np.float32)
_aTC = 2.0 * np.pi * ((_l[:, None] * _r[None, :]) % N10) / N10
_TWCR = np.cos(_aTC).astype(np.float32)
_TWCI = np.sin(_aTC).astype(np.float32)


def _dot(a, b):
    return jnp.dot(a, b, preferred_element_type=jnp.float32)


def _lowfft_body(low_ref, c5_ref, s5_ref, flr_ref, fli_ref):
    L = low_ref[0]
    C5 = c5_ref[...]
    S5 = s5_ref[...]
    t1r = _dot(C5, L)
    t1i = _dot(S5, L)
    flr_ref[0] = _dot(t1r, C5) - _dot(t1i, S5)
    fli_ref[0] = _dot(t1r, S5) + _dot(t1i, C5)


def _radix4(x0r, x0i, x1r, x1i, x2r, x2i, x3r, x3i):
    ar, ai = x0r + x2r, x0i + x2i
    br, bi = x0r - x2r, x0i - x2i
    cr, ci = x1r + x3r, x1i + x3i
    dr, di = x1r - x3r, x1i - x3i
    return (
        (ar + cr, ai + ci),
        (br - di, bi + dr),
        (ar - cr, ai - ci),
        (br + di, bi - dr),
    )


def _fused_body(ftr_ref, fti_ref, hbr_ref, hbi_ref, cb_ref, sb_ref,
                twbr_ref, twbi_ref, twcr_ref, twci_ref,
                out_ref, gr_ref, gi_ref):
    CB = cb_ref[...]
    SB = sb_ref[...]

    Fr = jnp.concatenate([ftr_ref[0], hbr_ref[0, TOP:, :]], axis=0)
    Fi = jnp.concatenate([fti_ref[0], hbi_ref[0, TOP:, :]], axis=0)
    slabs = _radix4(
        Fr[:, 0:256], Fi[:, 0:256], Fr[:, 256:512], Fi[:, 256:512],
        Fr[:, 512:768], Fi[:, 512:768], Fr[:, 768:1024], Fi[:, 768:1024])
    for n2 in range(R4):
        Zr, Zi = slabs[n2]
        if n2 > 0:
            tr = twbr_ref[n2:n2 + 1, :]
            ti = twbi_ref[n2:n2 + 1, :]
            Zr, Zi = Zr * tr - Zi * ti, Zr * ti + Zi * tr
        sl = slice(L1 * n2, L1 * (n2 + 1))
        gr_ref[:, sl] = _dot(Zr, CB) - _dot(Zi, SB)
        gi_ref[:, sl] = _dot(Zr, SB) + _dot(Zi, CB)

    cslabs = _radix4(
        gr_ref[0:256, :], gi_ref[0:256, :], gr_ref[256:512, :], gi_ref[256:512, :],
        gr_ref[512:768, :], gi_ref[512:768, :], gr_ref[768:1024, :], gi_ref[768:1024, :])
    for m2 in range(R4):
        Zr, Zi = cslabs[m2]
        if m2 > 0:
            tr = twcr_ref[:, m2:m2 + 1]
            ti = twci_ref[:, m2:m2 + 1]
            Zr, Zi = Zr * tr - Zi * ti, Zr * ti + Zi * tr
        out_ref[0, L1 * m2:L1 * (m2 + 1), :] = _dot(CB, Zr) - _dot(SB, Zi)


@functools.partial(jax.jit, static_argnums=())
def kernel(low_freq_image, hf_real, hf_imag, sel_indices):
    del sel_indices

    c5 = jnp.asarray(_C5)
    s5 = jnp.asarray(_S5)

    full_spec5 = pl.BlockSpec((N5, N5), lambda b: (0, 0))
    flr, fli = pl.pallas_call(
        _lowfft_body,
        grid=(B,),
        in_specs=[
            pl.BlockSpec((1, N5, N5), lambda b: (b, 0, 0)),
            full_spec5,
            full_spec5,
        ],
        out_specs=[
            pl.BlockSpec((1, N5, N5), lambda b: (b, 0, 0)),
            pl.BlockSpec((1, N5, N5), lambda b: (b, 0, 0)),
        ],
        out_shape=[
            jax.ShapeDtypeStruct((B, N5, N5), jnp.float32),
            jax.ShapeDtypeStruct((B, N5, N5), jnp.float32),
        ],
    )(low_freq_image, c5, s5)

    ftr = flr.reshape(B, TOP, N10)
    fti = fli.reshape(B, TOP, N10)

    hfr = hf_real.reshape(B, N10, N10)
    hfi = hf_imag.reshape(B, N10, N10)

    def _const(x):
        return pl.BlockSpec(x.shape, lambda b: tuple(0 for _ in x.shape))

    cb = jnp.asarray(_CB)
    sb = jnp.asarray(_SB)
    twbr = jnp.asarray(_TWBR)
    twbi = jnp.asarray(_TWBI)
    twcr = jnp.asarray(_TWCR)
    twci = jnp.asarray(_TWCI)

    out_s = pl.pallas_call(
        _fused_body,
        grid=(B,),
        in_specs=[
            pl.BlockSpec((1, TOP, N10), lambda b: (b, 0, 0)),
            pl.BlockSpec((1, TOP, N10), lambda b: (b, 0, 0)),
            pl.BlockSpec((1, N10, N10), lambda b: (b, 0, 0)),
            pl.BlockSpec((1, N10, N10), lambda b: (b, 0, 0)),
            _const(cb), _const(sb),
            _const(twbr), _const(twbi), _const(twcr), _const(twci),
        ],
        out_specs=pl.BlockSpec((1, N10, N10), lambda b: (b, 0, 0)),
        out_shape=jax.ShapeDtypeStruct((B, N10, N10), jnp.float32),
        scratch_shapes=[
            pltpu.VMEM((N10, N10), jnp.float32),
            pltpu.VMEM((N10, N10), jnp.float32),
        ],
    )(ftr, fti, hfr, hfi, cb, sb, twbr, twbi, twcr, twci)

    out = (out_s.reshape(B, R4, L1, R4, L1)
           .transpose(0, 2, 1, 4, 3)
           .reshape(B, N10, N10))
    return out


# speedup vs baseline: 1.5341x; 1.5341x over previous
import functools

import jax
import jax.numpy as jnp
import numpy as np
from jax.experimental import pallas as pl
from jax.experimental.pallas import tpu as pltpu

B = 8
N5 = 512
N10 = 1024
TOP = 256
NBOT = N10 - TOP

_k5 = np.arange(N5)
_a5 = 2.0 * np.pi * ((_k5[:, None] * _k5[None, :]) % N5) / N5
_C5 = np.cos(_a5).astype(np.float32)
_S5 = (-np.sin(_a5)).astype(np.float32)
_C5E, _C5O = _C5[0::2], _C5[1::2]
_S5E, _S5O = _S5[0::2], _S5[1::2]

_k = np.arange(N10)
_a = 2.0 * np.pi * ((_k[:, None] * _k[None, :]) % N10) / N10
_P = (np.cos(_a) / N10).astype(np.float32)
_Q = (np.sin(_a) / N10).astype(np.float32)


def _dot(a, b):
    return jnp.dot(a, b, preferred_element_type=jnp.float32)


def _lowfft_body(low_ref, c5_ref, s5_ref, c5e_ref, s5e_ref, c5o_ref, s5o_ref,
                 ftr_ref, fti_ref):
    L = low_ref[0]
    C5 = c5_ref[...]
    S5 = s5_ref[...]
    ur = _dot(L, C5)
    ui = _dot(L, S5)
    C5e = c5e_ref[...]
    S5e = s5e_ref[...]
    ftr_ref[0, :, 0:N5] = _dot(C5e, ur) - _dot(S5e, ui)
    fti_ref[0, :, 0:N5] = _dot(C5e, ui) + _dot(S5e, ur)
    C5o = c5o_ref[...]
    S5o = s5o_ref[...]
    ftr_ref[0, :, N5:N10] = _dot(C5o, ur) - _dot(S5o, ui)
    fti_ref[0, :, N5:N10] = _dot(C5o, ui) + _dot(S5o, ur)


def _fused_body(ftr_ref, fti_ref, hbr_ref, hbi_ref, p_ref, q_ref,
                out_ref, gr_ref, gi_ref):
    P = p_ref[...]
    Q = q_ref[...]
    Ftr = ftr_ref[0]
    Fti = fti_ref[0]
    gr_ref[:TOP] = _dot(Ftr, P) - _dot(Fti, Q)
    gi_ref[:TOP] = _dot(Ftr, Q) + _dot(Fti, P)
    Fbr = hbr_ref[0]
    Fbi = hbi_ref[0]
    gr_ref[TOP:] = _dot(Fbr, P) - _dot(Fbi, Q)
    gi_ref[TOP:] = _dot(Fbr, Q) + _dot(Fbi, P)
    out_ref[0] = _dot(P, gr_ref[...]) - _dot(Q, gi_ref[...])


@functools.partial(jax.jit, static_argnums=())
def kernel(low_freq_image, hf_real, hf_imag, sel_indices):
    del sel_indices

    c5 = jnp.asarray(_C5)
    s5 = jnp.asarray(_S5)
    c5e = jnp.asarray(_C5E)
    s5e = jnp.asarray(_S5E)
    c5o = jnp.asarray(_C5O)
    s5o = jnp.asarray(_S5O)

    full5 = pl.BlockSpec((N5, N5), lambda b: (0, 0))
    half5 = pl.BlockSpec((TOP, N5), lambda b: (0, 0))
    ftr, fti = pl.pallas_call(
        _lowfft_body,
        grid=(B,),
        in_specs=[
            pl.BlockSpec((1, N5, N5), lambda b: (b, 0, 0)),
            full5, full5, half5, half5, half5, half5,
        ],
        out_specs=[
            pl.BlockSpec((1, TOP, N10), lambda b: (b, 0, 0)),
            pl.BlockSpec((1, TOP, N10), lambda b: (b, 0, 0)),
        ],
        out_shape=[
            jax.ShapeDtypeStruct((B, TOP, N10), jnp.float32),
            jax.ShapeDtypeStruct((B, TOP, N10), jnp.float32),
        ],
    )(low_freq_image, c5, s5, c5e, s5e, c5o, s5o)

    hbr = hf_real[:, TOP * N10:].reshape(B, NBOT, N10)
    hbi = hf_imag[:, TOP * N10:].reshape(B, NBOT, N10)

    p = jnp.asarray(_P)
    q = jnp.asarray(_Q)
    full10 = pl.BlockSpec((N10, N10), lambda b: (0, 0))

    out = pl.pallas_call(
        _fused_body,
        grid=(B,),
        in_specs=[
            pl.BlockSpec((1, TOP, N10), lambda b: (b, 0, 0)),
            pl.BlockSpec((1, TOP, N10), lambda b: (b, 0, 0)),
            pl.BlockSpec((1, NBOT, N10), lambda b: (b, 0, 0)),
            pl.BlockSpec((1, NBOT, N10), lambda b: (b, 0, 0)),
            full10, full10,
        ],
        out_specs=pl.BlockSpec((1, N10, N10), lambda b: (b, 0, 0)),
        out_shape=jax.ShapeDtypeStruct((B, N10, N10), jnp.float32),
        scratch_shapes=[
            pltpu.VMEM((N10, N10), jnp.float32),
            pltpu.VMEM((N10, N10), jnp.float32),
        ],
    )(ftr, fti, hbr, hbi, p, q)

    return out


# speedup vs baseline: 1.8336x; 1.1952x over previous
import functools

import jax
import jax.numpy as jnp
import numpy as np
from jax.experimental import pallas as pl
from jax.experimental.pallas import tpu as pltpu

B = 8
N5 = 512
N10 = 1024
TOP = 256
NBOT = N10 - TOP

_k5 = np.arange(N5)
_a5 = 2.0 * np.pi * ((_k5[:, None] * _k5[None, :]) % N5) / N5
_C5 = np.cos(_a5).astype(np.float32)
_S5 = (-np.sin(_a5)).astype(np.float32)
_C5E, _C5O = _C5[0::2], _C5[1::2]
_S5E, _S5O = _S5[0::2], _S5[1::2]

_k = np.arange(N10)
_a = 2.0 * np.pi * ((_k[:, None] * _k[None, :]) % N10) / N10
_P = (np.cos(_a) / N10).astype(jnp.bfloat16)
_Q = (np.sin(_a) / N10).astype(jnp.bfloat16)


def _dot(a, b):
    return jnp.dot(a, b, preferred_element_type=jnp.float32)


def _lowfft_body(low_ref, c5_ref, s5_ref, c5e_ref, s5e_ref, c5o_ref, s5o_ref,
                 ftr_ref, fti_ref):
    L = low_ref[0]
    C5 = c5_ref[...]
    S5 = s5_ref[...]
    ur = _dot(L, C5)
    ui = _dot(L, S5)
    C5e = c5e_ref[...]
    S5e = s5e_ref[...]
    ftr_ref[0, :, 0:N5] = (_dot(C5e, ur) - _dot(S5e, ui)).astype(jnp.bfloat16)
    fti_ref[0, :, 0:N5] = (_dot(C5e, ui) + _dot(S5e, ur)).astype(jnp.bfloat16)
    C5o = c5o_ref[...]
    S5o = s5o_ref[...]
    ftr_ref[0, :, N5:N10] = (_dot(C5o, ur) - _dot(S5o, ui)).astype(jnp.bfloat16)
    fti_ref[0, :, N5:N10] = (_dot(C5o, ui) + _dot(S5o, ur)).astype(jnp.bfloat16)


def _fused_body(ftr_ref, fti_ref, hbr_ref, hbi_ref, p_ref, q_ref,
                out_ref, gr_ref, gi_ref):
    P = p_ref[...]
    Q = q_ref[...]
    Ftr = ftr_ref[0]
    Fti = fti_ref[0]
    gr_ref[:TOP] = _dot(Ftr, P) - _dot(Fti, Q)
    gi_ref[:TOP] = _dot(Ftr, Q) + _dot(Fti, P)
    Fbr = hbr_ref[0]
    Fbi = hbi_ref[0]
    gr_ref[TOP:] = _dot(Fbr, P) - _dot(Fbi, Q)
    gi_ref[TOP:] = _dot(Fbr, Q) + _dot(Fbi, P)
    out_ref[0] = _dot(P, gr_ref[...]) - _dot(Q, gi_ref[...])


@functools.partial(jax.jit, static_argnums=())
def kernel(low_freq_image, hf_real, hf_imag, sel_indices):
    del sel_indices

    c5 = jnp.asarray(_C5)
    s5 = jnp.asarray(_S5)
    c5e = jnp.asarray(_C5E)
    s5e = jnp.asarray(_S5E)
    c5o = jnp.asarray(_C5O)
    s5o = jnp.asarray(_S5O)

    full5 = pl.BlockSpec((N5, N5), lambda b: (0, 0))
    half5 = pl.BlockSpec((TOP, N5), lambda b: (0, 0))
    ftr, fti = pl.pallas_call(
        _lowfft_body,
        grid=(B,),
        in_specs=[
            pl.BlockSpec((1, N5, N5), lambda b: (b, 0, 0)),
            full5, full5, half5, half5, half5, half5,
        ],
        out_specs=[
            pl.BlockSpec((1, TOP, N10), lambda b: (b, 0, 0)),
            pl.BlockSpec((1, TOP, N10), lambda b: (b, 0, 0)),
        ],
        out_shape=[
            jax.ShapeDtypeStruct((B, TOP, N10), jnp.bfloat16),
            jax.ShapeDtypeStruct((B, TOP, N10), jnp.bfloat16),
        ],
    )(low_freq_image, c5, s5, c5e, s5e, c5o, s5o)

    hbr = hf_real[:, TOP * N10:].astype(jnp.bfloat16).reshape(B, NBOT, N10)
    hbi = hf_imag[:, TOP * N10:].astype(jnp.bfloat16).reshape(B, NBOT, N10)

    p = jnp.asarray(_P)
    q = jnp.asarray(_Q)
    full10 = pl.BlockSpec((N10, N10), lambda b: (0, 0))

    out = pl.pallas_call(
        _fused_body,
        grid=(B,),
        in_specs=[
            pl.BlockSpec((1, TOP, N10), lambda b: (b, 0, 0)),
            pl.BlockSpec((1, TOP, N10), lambda b: (b, 0, 0)),
            pl.BlockSpec((1, NBOT, N10), lambda b: (b, 0, 0)),
            pl.BlockSpec((1, NBOT, N10), lambda b: (b, 0, 0)),
            full10, full10,
        ],
        out_specs=pl.BlockSpec((1, N10, N10), lambda b: (b, 0, 0)),
        out_shape=jax.ShapeDtypeStruct((B, N10, N10), jnp.float32),
        scratch_shapes=[
            pltpu.VMEM((N10, N10), jnp.float32),
            pltpu.VMEM((N10, N10), jnp.float32),
        ],
    )(ftr, fti, hbr, hbi, p, q)

    return out


# speedup vs baseline: 1.8813x; 1.0260x over previous
import functools

import jax
import jax.numpy as jnp
import numpy as np
from jax.experimental import pallas as pl
from jax.experimental.pallas import tpu as pltpu

B = 8
N5 = 512
N10 = 1024
TOP = 256
NBOT = N10 - TOP

_k5 = np.arange(N5)
_a5 = 2.0 * np.pi * ((_k5[:, None] * _k5[None, :]) % N5) / N5
_C5 = np.cos(_a5).astype(np.float32)
_S5 = (-np.sin(_a5)).astype(np.float32)
_C5E, _C5O = _C5[0::2], _C5[1::2]
_S5E, _S5O = _S5[0::2], _S5[1::2]

_k = np.arange(N10)
_a = 2.0 * np.pi * ((_k[:, None] * _k[None, :]) % N10) / N10
_P = (np.cos(_a) / N10).astype(jnp.bfloat16)
_Q = (np.sin(_a) / N10).astype(jnp.bfloat16)


def _dot(a, b):
    return jnp.dot(a, b, preferred_element_type=jnp.float32)


def _fused_body(low_ref, hbr_ref, hbi_ref, c5_ref, s5_ref,
                c5e_ref, s5e_ref, c5o_ref, s5o_ref, p_ref, q_ref,
                out_ref, gr_ref, gi_ref):
    L = low_ref[0]
    C5 = c5_ref[...]
    S5 = s5_ref[...]
    ur = _dot(L, C5)
    ui = _dot(L, S5)
    C5e = c5e_ref[...]
    S5e = s5e_ref[...]
    C5o = c5o_ref[...]
    S5o = s5o_ref[...]
    Ftr = jnp.concatenate(
        [_dot(C5e, ur) - _dot(S5e, ui), _dot(C5o, ur) - _dot(S5o, ui)],
        axis=1).astype(jnp.bfloat16)
    Fti = jnp.concatenate(
        [_dot(C5e, ui) + _dot(S5e, ur), _dot(C5o, ui) + _dot(S5o, ur)],
        axis=1).astype(jnp.bfloat16)

    P = p_ref[...]
    Q = q_ref[...]
    gr_ref[:TOP] = (_dot(Ftr, P) - _dot(Fti, Q)).astype(jnp.bfloat16)
    gi_ref[:TOP] = (_dot(Ftr, Q) + _dot(Fti, P)).astype(jnp.bfloat16)
    Fbr = hbr_ref[0]
    Fbi = hbi_ref[0]
    gr_ref[TOP:] = (_dot(Fbr, P) - _dot(Fbi, Q)).astype(jnp.bfloat16)
    gi_ref[TOP:] = (_dot(Fbr, Q) + _dot(Fbi, P)).astype(jnp.bfloat16)

    out_ref[0] = _dot(P, gr_ref[...]) - _dot(Q, gi_ref[...])


@functools.partial(jax.jit, static_argnums=())
def kernel(low_freq_image, hf_real, hf_imag, sel_indices):
    del sel_indices

    c5 = jnp.asarray(_C5)
    s5 = jnp.asarray(_S5)
    c5e = jnp.asarray(_C5E)
    s5e = jnp.asarray(_S5E)
    c5o = jnp.asarray(_C5O)
    s5o = jnp.asarray(_S5O)

    hbr = hf_real[:, TOP * N10:].astype(jnp.bfloat16).reshape(B, NBOT, N10)
    hbi = hf_imag[:, TOP * N10:].astype(jnp.bfloat16).reshape(B, NBOT, N10)

    p = jnp.asarray(_P)
    q = jnp.asarray(_Q)
    full5 = pl.BlockSpec((N5, N5), lambda b: (0, 0))
    half5 = pl.BlockSpec((TOP, N5), lambda b: (0, 0))
    full10 = pl.BlockSpec((N10, N10), lambda b: (0, 0))

    out = pl.pallas_call(
        _fused_body,
        grid=(B,),
        in_specs=[
            pl.BlockSpec((1, N5, N5), lambda b: (b, 0, 0)),
            pl.BlockSpec((1, NBOT, N10), lambda b: (b, 0, 0)),
            pl.BlockSpec((1, NBOT, N10), lambda b: (b, 0, 0)),
            full5, full5, half5, half5, half5, half5,
            full10, full10,
        ],
        out_specs=pl.BlockSpec((1, N10, N10), lambda b: (b, 0, 0)),
        out_shape=jax.ShapeDtypeStruct((B, N10, N10), jnp.float32),
        scratch_shapes=[
            pltpu.VMEM((N10, N10), jnp.bfloat16),
            pltpu.VMEM((N10, N10), jnp.bfloat16),
        ],
    )(low_freq_image, hbr, hbi, c5, s5, c5e, s5e, c5o, s5o, p, q)

    return out
